# baseline (speedup 1.0000x reference)
import functools

import jax
import jax.numpy as jnp
from jax import lax
from jax.experimental import pallas as pl
from jax.experimental.pallas import tpu as pltpu
from jax.experimental.pallas import tpu_sc as plsc

ROWS, COLS = 2048, 4096
HCOLS = COLS // 2
GROUP = 64
NLEV = 16
NELEM = ROWS * COLS
NGROUPS = NELEM // GROUP
NWORKERS = 32
CROWS = 8
CELEM = CROWS * COLS
CGROUP = CELEM // GROUP
NCHUNK = (ROWS // CROWS) // NWORKERS


def _sc_body(
    x_hbm,
    lv_hbm,
    out_hbm,
    xbuf0,
    xbuf1,
    lvbuf0,
    lvbuf1,
    obufa,
    obufb,
    sin0,
    sin1,
    souta,
    soutb,
):
    info = plsc.get_sparse_core_info()
    nc = info.num_cores
    wid = lax.axis_index("s") * nc + lax.axis_index("c")

    ii = lax.iota(jnp.int32, 16)
    zero = ii * 0
    one = zero + 1
    two = zero + 2
    d = (
        jnp.where(ii >= 1, one, zero)
        + jnp.where(ii >= 3, one, zero)
        + jnp.where(ii >= 7, one, zero)
        + jnp.where(ii >= 15, one, zero)
    )
    heap = (ii - lax.shift_left(one, d) + 1) * lax.shift_right_logical(
        zero + 16, d
    ) + lax.shift_right_logical(zero + 8, d) - 1
    heap = jnp.maximum(heap, zero)
    heap_p1 = heap + one
    fifteen = zero + 15

    def take16(vec, idx):
        return vec.at[idx].get(mode="promise_in_bounds")

    xbufs = (xbuf0, xbuf1)
    lvbufs = (lvbuf0, lvbuf1)
    sins = (sin0, sin1)

    def start_in(c, half):
        tr = wid * NCHUNK + c
        pltpu.async_copy(
            x_hbm.at[pl.ds(tr * CROWS, CROWS), :], xbufs[half], sins[half]
        )
        pltpu.async_copy(
            lv_hbm.at[pl.ds(tr * CROWS, CROWS), :], lvbufs[half], sins[half]
        )

    def wait_in(half):
        pltpu.make_async_copy(
            x_hbm.at[pl.ds(0, CROWS), :], xbufs[half], sins[half]
        ).wait()
        pltpu.make_async_copy(
            lv_hbm.at[pl.ds(0, CROWS), :], lvbufs[half], sins[half]
        ).wait()

    def wait_out(obuf, sout):
        pltpu.make_async_copy(
            out_hbm.at[pl.ds(0, CROWS), pl.ds(0, HCOLS)], obuf, sout
        ).wait()

    def compute_half(half, part):
        xbuf, lvbuf = xbufs[half], lvbufs[half]
        obuf = obufa if part == 0 else obufb

        @plsc.parallel_loop(0, CGROUP // 2, unroll=4)
        def _search(j):
            r = j >> 5
            gq = j & 31
            lv = lvbuf[r, pl.ds((part * (HCOLS // GROUP) + gq) * NLEV, NLEV)]
            bvec = (take16(lv, heap) + take16(lv, heap_p1)) * 0.5
            for k in range(GROUP // 16):
                cq = gq * GROUP + k * 16
                xv = xbuf[r, pl.ds(part * HCOLS + cq, 16)]
                pos = zero
                for _ in range(4):
                    bv = take16(bvec, pos)
                    step = jnp.where(xv > bv, two, one)
                    pos = pos + pos + step
                obuf[r, pl.ds(cq, 16)] = take16(lv, pos - fifteen)

    start_in(0, 0)

    def pair_body(k, carry):
        for half in range(2):
            c = 2 * k + half
            tr = wid * NCHUNK + c
            wait_in(half)

            @pl.when(c + 1 < NCHUNK)
            def _():
                start_in(c + 1, 1 - half)

            @pl.when(c > 0)
            def _():
                wait_out(obufa, souta)

            compute_half(half, 0)
            pltpu.async_copy(
                obufa,
                out_hbm.at[pl.ds(tr * CROWS, CROWS), pl.ds(0, HCOLS)],
                souta,
            )

            @pl.when(c > 0)
            def _():
                wait_out(obufb, soutb)

            compute_half(half, 1)
            pltpu.async_copy(
                obufb,
                out_hbm.at[pl.ds(tr * CROWS, CROWS), pl.ds(HCOLS, HCOLS)],
                soutb,
            )
        return carry

    lax.fori_loop(0, NCHUNK // 2, pair_body, 0)
    wait_out(obufa, souta)
    wait_out(obufb, soutb)


@jax.jit
def _quantize(x, lf):
    mesh = plsc.VectorSubcoreMesh(core_axis_name="c", subcore_axis_name="s")
    return pl.kernel(
        _sc_body,
        out_type=jax.ShapeDtypeStruct((ROWS, COLS), jnp.float32),
        mesh=mesh,
        scratch_types=[
            pltpu.VMEM((CROWS, COLS), jnp.float32),
            pltpu.VMEM((CROWS, COLS), jnp.float32),
            pltpu.VMEM((CROWS, CGROUP * NLEV // CROWS), jnp.float32),
            pltpu.VMEM((CROWS, CGROUP * NLEV // CROWS), jnp.float32),
            pltpu.VMEM((CROWS, HCOLS), jnp.float32),
            pltpu.VMEM((CROWS, HCOLS), jnp.float32),
            pltpu.SemaphoreType.DMA,
            pltpu.SemaphoreType.DMA,
            pltpu.SemaphoreType.DMA,
            pltpu.SemaphoreType.DMA,
        ],
        compiler_params=pltpu.CompilerParams(
            needs_layout_passes=False, use_tc_tiling_on_sc=True
        ),
    )(x, lf)


def kernel(x, levels):
    return _quantize(x, levels.reshape(ROWS, NELEM // GROUP * NLEV // ROWS))


# speedup vs baseline: 1.1031x; 1.1031x over previous
import functools

import jax
import jax.numpy as jnp
from jax import lax
from jax.experimental import pallas as pl
from jax.experimental.pallas import tpu as pltpu
from jax.experimental.pallas import tpu_sc as plsc

ROWS, COLS = 2048, 4096
HCOLS = COLS // 2
GROUP = 64
NLEV = 16
NELEM = ROWS * COLS
NGROUPS = NELEM // GROUP
NWORKERS = 32
CROWS = 8
CELEM = CROWS * COLS
CGROUP = CELEM // GROUP
NCHUNK = (ROWS // CROWS) // NWORKERS


def _sc_body(
    x_hbm,
    lv_hbm,
    out_hbm,
    xbuf0,
    xbuf1,
    lvbuf0,
    lvbuf1,
    obufa,
    obufb,
    sin0,
    sin1,
    souta,
    soutb,
):
    info = plsc.get_sparse_core_info()
    nc = info.num_cores
    wid = lax.axis_index("s") * nc + lax.axis_index("c")

    ii = lax.iota(jnp.int32, 16)
    zero = ii * 0
    one = zero + 1
    two = zero + 2
    d = (
        jnp.where(ii >= 1, one, zero)
        + jnp.where(ii >= 3, one, zero)
        + jnp.where(ii >= 7, one, zero)
        + jnp.where(ii >= 15, one, zero)
    )
    heap = (ii - lax.shift_left(one, d) + 1) * lax.shift_right_logical(
        zero + 16, d
    ) + lax.shift_right_logical(zero + 8, d) - 1
    heap = jnp.maximum(heap, zero)
    heap_p1 = heap + one
    fifteen = zero + 15

    def take16(vec, idx):
        return vec.at[idx].get(mode="promise_in_bounds")

    xbufs = (xbuf0, xbuf1)
    lvbufs = (lvbuf0, lvbuf1)
    sins = (sin0, sin1)

    def start_in(c, half):
        tr = wid * NCHUNK + c
        pltpu.async_copy(
            x_hbm.at[pl.ds(tr * CROWS, CROWS), :], xbufs[half], sins[half]
        )
        pltpu.async_copy(
            lv_hbm.at[pl.ds(tr * CGROUP * NLEV, CGROUP * NLEV)],
            lvbufs[half],
            sins[half],
        )

    def wait_in(half):
        pltpu.make_async_copy(
            x_hbm.at[pl.ds(0, CROWS), :], xbufs[half], sins[half]
        ).wait()
        pltpu.make_async_copy(
            lv_hbm.at[pl.ds(0, CGROUP * NLEV)], lvbufs[half], sins[half]
        ).wait()

    def wait_out(obuf, sout):
        pltpu.make_async_copy(
            out_hbm.at[pl.ds(0, CROWS), pl.ds(0, HCOLS)], obuf, sout
        ).wait()

    def compute_half(half, part):
        xbuf, lvbuf = xbufs[half], lvbufs[half]
        obuf = obufa if part == 0 else obufb

        @plsc.parallel_loop(0, CGROUP // 2, unroll=8)
        def _search(j):
            r = j >> 5
            gq = j & 31
            g = r * (COLS // GROUP) + part * (HCOLS // GROUP) + gq
            lv = lvbuf[pl.ds(g * NLEV, NLEV)]
            bvec = (take16(lv, heap) + take16(lv, heap_p1)) * 0.5
            for k in range(GROUP // 16):
                cq = gq * GROUP + k * 16
                xv = xbuf[r, pl.ds(part * HCOLS + cq, 16)]
                pos = zero
                for _ in range(4):
                    bv = take16(bvec, pos)
                    step = jnp.where(xv > bv, two, one)
                    pos = pos + pos + step
                obuf[r, pl.ds(cq, 16)] = take16(lv, pos - fifteen)

    start_in(0, 0)

    def pair_body(k, carry):
        for half in range(2):
            c = 2 * k + half
            tr = wid * NCHUNK + c
            wait_in(half)

            @pl.when(c + 1 < NCHUNK)
            def _():
                start_in(c + 1, 1 - half)

            @pl.when(c > 0)
            def _():
                wait_out(obufa, souta)

            compute_half(half, 0)
            pltpu.async_copy(
                obufa,
                out_hbm.at[pl.ds(tr * CROWS, CROWS), pl.ds(0, HCOLS)],
                souta,
            )

            @pl.when(c > 0)
            def _():
                wait_out(obufb, soutb)

            compute_half(half, 1)
            pltpu.async_copy(
                obufb,
                out_hbm.at[pl.ds(tr * CROWS, CROWS), pl.ds(HCOLS, HCOLS)],
                soutb,
            )
        return carry

    lax.fori_loop(0, NCHUNK // 2, pair_body, 0)
    wait_out(obufa, souta)
    wait_out(obufb, soutb)


@jax.jit
def _quantize(x, lf):
    mesh = plsc.VectorSubcoreMesh(core_axis_name="c", subcore_axis_name="s")
    return pl.kernel(
        _sc_body,
        out_type=jax.ShapeDtypeStruct((ROWS, COLS), jnp.float32),
        mesh=mesh,
        scratch_types=[
            pltpu.VMEM((CROWS, COLS), jnp.float32),
            pltpu.VMEM((CROWS, COLS), jnp.float32),
            pltpu.VMEM((CGROUP * NLEV,), jnp.float32),
            pltpu.VMEM((CGROUP * NLEV,), jnp.float32),
            pltpu.VMEM((CROWS, HCOLS), jnp.float32),
            pltpu.VMEM((CROWS, HCOLS), jnp.float32),
            pltpu.SemaphoreType.DMA,
            pltpu.SemaphoreType.DMA,
            pltpu.SemaphoreType.DMA,
            pltpu.SemaphoreType.DMA,
        ],
        compiler_params=pltpu.CompilerParams(
            needs_layout_passes=False, use_tc_tiling_on_sc=True
        ),
    )(x, lf)


def kernel(x, levels):
    return _quantize(x, levels.reshape(-1))


# speedup vs baseline: 1.1555x; 1.0475x over previous
import functools

import jax
import jax.numpy as jnp
from jax import lax
from jax.experimental import pallas as pl
from jax.experimental.pallas import tpu as pltpu
from jax.experimental.pallas import tpu_sc as plsc

ROWS, COLS = 2048, 4096
HCOLS = COLS // 2
GROUP = 64
NLEV = 16
NELEM = ROWS * COLS
NGROUPS = NELEM // GROUP
NWORKERS = 32
CROWS = 8
CELEM = CROWS * COLS
CGROUP = CELEM // GROUP
NCHUNK = (ROWS // CROWS) // NWORKERS


def _sc_body(
    x_hbm,
    lv_hbm,
    out_hbm,
    xbuf0,
    xbuf1,
    lvbuf0,
    lvbuf1,
    obufa,
    obufb,
    sin0,
    sin1,
    souta,
    soutb,
):
    info = plsc.get_sparse_core_info()
    nc = info.num_cores
    wid = lax.axis_index("s") * nc + lax.axis_index("c")

    ii = lax.iota(jnp.int32, 16)
    zero = ii * 0
    one = zero + 1
    two = zero + 2
    d = (
        jnp.where(ii >= 1, one, zero)
        + jnp.where(ii >= 3, one, zero)
        + jnp.where(ii >= 7, one, zero)
        + jnp.where(ii >= 15, one, zero)
    )
    heap = (ii - lax.shift_left(one, d) + 1) * lax.shift_right_logical(
        zero + 16, d
    ) + lax.shift_right_logical(zero + 8, d) - 1
    heap = jnp.maximum(heap, zero)
    heap_p1 = heap + one
    fifteen = zero + 15

    def take16(vec, idx):
        return vec.at[idx].get(mode="promise_in_bounds")

    xbufs = (xbuf0, xbuf1)
    lvbufs = (lvbuf0, lvbuf1)
    sins = (sin0, sin1)

    def start_in(c, half):
        tr = wid * NCHUNK + c
        pltpu.async_copy(
            x_hbm.at[pl.ds(tr * CROWS, CROWS), :], xbufs[half], sins[half]
        )
        pltpu.async_copy(
            lv_hbm.at[pl.ds(tr * CGROUP * NLEV, CGROUP * NLEV)],
            lvbufs[half],
            sins[half],
        )

    def wait_in(half):
        pltpu.make_async_copy(
            x_hbm.at[pl.ds(0, CROWS), :], xbufs[half], sins[half]
        ).wait()
        pltpu.make_async_copy(
            lv_hbm.at[pl.ds(0, CGROUP * NLEV)], lvbufs[half], sins[half]
        ).wait()

    def wait_out(obuf, sout):
        pltpu.make_async_copy(
            out_hbm.at[pl.ds(0, CROWS), pl.ds(0, HCOLS)], obuf, sout
        ).wait()

    def compute_half(half, part):
        xbuf, lvbuf = xbufs[half], lvbufs[half]
        obuf = obufa if part == 0 else obufb

        @plsc.parallel_loop(0, CGROUP // 2, unroll=4)
        def _search(j):
            r = j >> 5
            gq = j & 31
            g = r * (COLS // GROUP) + part * (HCOLS // GROUP) + gq
            lv = lvbuf[pl.ds(g * NLEV, NLEV)]
            bvec = (take16(lv, heap) + take16(lv, heap_p1)) * 0.5
            for k in range(GROUP // 16):
                cq = gq * GROUP + k * 16
                xv = xbuf[r, pl.ds(part * HCOLS + cq, 16)]
                pos = zero
                for _ in range(4):
                    bv = take16(bvec, pos)
                    step = jnp.where(xv > bv, two, one)
                    pos = pos + pos + step
                obuf[r, pl.ds(cq, 16)] = take16(lv, pos - fifteen)

    start_in(0, 0)

    def pair_body(k, carry):
        for half in range(2):
            c = 2 * k + half
            tr = wid * NCHUNK + c
            wait_in(half)

            @pl.when(c + 1 < NCHUNK)
            def _():
                start_in(c + 1, 1 - half)

            @pl.when(c > 0)
            def _():
                wait_out(obufa, souta)

            compute_half(half, 0)
            pltpu.async_copy(
                obufa,
                out_hbm.at[pl.ds(tr * CROWS, CROWS), pl.ds(0, HCOLS)],
                souta,
            )

            @pl.when(c > 0)
            def _():
                wait_out(obufb, soutb)

            compute_half(half, 1)
            pltpu.async_copy(
                obufb,
                out_hbm.at[pl.ds(tr * CROWS, CROWS), pl.ds(HCOLS, HCOLS)],
                soutb,
            )
        return carry

    lax.fori_loop(0, NCHUNK // 2, pair_body, 0)
    wait_out(obufa, souta)
    wait_out(obufb, soutb)


@jax.jit
def _quantize(x, lf):
    mesh = plsc.VectorSubcoreMesh(core_axis_name="c", subcore_axis_name="s")
    return pl.kernel(
        _sc_body,
        out_type=jax.ShapeDtypeStruct((ROWS, COLS), jnp.float32),
        mesh=mesh,
        scratch_types=[
            pltpu.VMEM((CROWS, COLS), jnp.float32),
            pltpu.VMEM((CROWS, COLS), jnp.float32),
            pltpu.VMEM((CGROUP * NLEV,), jnp.float32),
            pltpu.VMEM((CGROUP * NLEV,), jnp.float32),
            pltpu.VMEM((CROWS, HCOLS), jnp.float32),
            pltpu.VMEM((CROWS, HCOLS), jnp.float32),
            pltpu.SemaphoreType.DMA,
            pltpu.SemaphoreType.DMA,
            pltpu.SemaphoreType.DMA,
            pltpu.SemaphoreType.DMA,
        ],
        compiler_params=pltpu.CompilerParams(
            needs_layout_passes=False, use_tc_tiling_on_sc=True
        ),
    )(x, lf)


def kernel(x, levels):
    return _quantize(x, levels.reshape(-1))
